# baseline (speedup 1.0000x reference)
import jax
from jax.experimental import pallas as pl

_GRID = 15
_VROWS = 8
_IROWS = 16


def _copy_kernel(v_ref, i_ref, vo_ref, io_ref):
    vo_ref[...] = v_ref[...]
    io_ref[...] = i_ref[...]


def kernel(vertices, indices):
    v2 = vertices.reshape(120, 2500)
    i2 = indices.reshape(240, 2500)
    vo, io = pl.pallas_call(
        _copy_kernel,
        grid=(_GRID,),
        out_shape=(
            jax.ShapeDtypeStruct(v2.shape, v2.dtype),
            jax.ShapeDtypeStruct(i2.shape, i2.dtype),
        ),
        in_specs=[
            pl.BlockSpec((_VROWS, 2500), lambda j: (j, 0)),
            pl.BlockSpec((_IROWS, 2500), lambda j: (j, 0)),
        ],
        out_specs=(
            pl.BlockSpec((_VROWS, 2500), lambda j: (j, 0)),
            pl.BlockSpec((_IROWS, 2500), lambda j: (j, 0)),
        ),
    )(v2, i2)
    return vo.reshape(vertices.shape), io.reshape(indices.shape)


# speedup vs baseline: 1.2133x; 1.2133x over previous
import jax
from jax.experimental import pallas as pl

_GRID = 25
_VROWS = 24
_IROWS = 48


def _copy_kernel(v_ref, i_ref, vo_ref, io_ref):
    vo_ref[...] = v_ref[...]
    io_ref[...] = i_ref[...]


def kernel(vertices, indices):
    v2 = vertices.reshape(600, 500)
    i2 = indices.reshape(1200, 500)
    vo, io = pl.pallas_call(
        _copy_kernel,
        grid=(_GRID,),
        out_shape=(
            jax.ShapeDtypeStruct(v2.shape, v2.dtype),
            jax.ShapeDtypeStruct(i2.shape, i2.dtype),
        ),
        in_specs=[
            pl.BlockSpec((_VROWS, 500), lambda j: (j, 0)),
            pl.BlockSpec((_IROWS, 500), lambda j: (j, 0)),
        ],
        out_specs=(
            pl.BlockSpec((_VROWS, 500), lambda j: (j, 0)),
            pl.BlockSpec((_IROWS, 500), lambda j: (j, 0)),
        ),
    )(v2, i2)
    return vo.reshape(vertices.shape), io.reshape(indices.shape)
